# Initial kernel scaffold; baseline (speedup 1.0000x reference)
import jax
import jax.numpy as jnp
from jax.experimental import pallas as pl


def kernel(x, batch, repeat_unit_mask, gate_w, gate_b):
    raise NotImplementedError("write your pallas kernel here")


# speedup vs baseline: 7.2566x; 7.2566x over previous
import functools

import jax
import jax.numpy as jnp
from jax import lax
from jax.experimental import pallas as pl
from jax.experimental.pallas import tpu as pltpu
from jax.experimental.pallas import tpu_sc as plsc

N_ROWS = 100000
DIM = 256
G = 1024
NC = 2
NS = 16
L = 16
NW = NC * NS
GPW = G // NW
CH = 128
NCOL = DIM // L
BPAD = 48


def _sc_body(x_hbm, batch_hbm, mask_hbm, w_hbm, b_hbm, bounds_hbm, out_hbm,
             xbuf0, bidx0, midx0, xbuf1, bidx1, midx1,
             wbuf, bbuf, boundsbuf, accbuf, sem0, sem1):
    cid = lax.axis_index("c")
    sid = lax.axis_index("s")
    wid = cid * NS + sid
    iot = lax.iota(jnp.int32, L)
    zero16 = jnp.zeros((L,), jnp.float32)

    pltpu.sync_copy(w_hbm, wbuf)
    pltpu.sync_copy(b_hbm, bbuf)
    pltpu.sync_copy(bounds_hbm, boundsbuf)

    def _zrow(r, _):
        for cc in range(NCOL):
            accbuf[r, pl.ds(cc * L, L)] = zero16
        return 0
    lax.fori_loop(0, GPW, _zrow, 0)

    def _scalar_at(ref, i):
        return plsc.load_gather(ref, [jnp.full((L,), i, jnp.int32)])[0]

    r_lo = _scalar_at(boundsbuf, wid)
    r_hi = _scalar_at(boundsbuf, wid + 1)
    start0 = (r_lo // CH) * CH
    nch = jnp.maximum((r_hi - start0 + CH - 1) // CH, 0)
    nchp = ((nch + 1) // 2) * 2

    wregs = [wbuf[pl.ds(cc * L, L)] for cc in range(NCOL)]
    bvec = bbuf[...]
    g_base = wid * GPW

    def _chunk_start(k):
        return jnp.minimum(start0 + k * CH, N_ROWS - CH)

    def _copies(k, bufs, sem):
        s = _chunk_start(k)
        xb, bb, mb = bufs
        return (pltpu.make_async_copy(x_hbm.at[pl.ds(s, CH)], xb, sem),
                pltpu.make_async_copy(batch_hbm.at[pl.ds(s, CH)], bb, sem),
                pltpu.make_async_copy(mask_hbm.at[pl.ds(s, CH)], mb, sem))

    def _start(k, bufs, sem):
        for d in _copies(k, bufs, sem):
            d.start()

    def _wait(k, bufs, sem):
        for d in _copies(k, bufs, sem):
            d.wait()

    def _flush(cur_g, denv, accs):
        den = jnp.where(denv > 0.0, denv, 1.0)
        rec = 1.0 / den
        gl = cur_g - g_base
        for cc in range(NCOL):
            accbuf[gl, pl.ds(cc * L, L)] = accs[cc] * rec

    def _process(k, bufs, carry):
        xb, bb, mb = bufs
        s = _chunk_start(k)
        base = start0 + k * CH

        def row_body(r, c2):
            cur_g = c2[0]
            denv = c2[1]
            accs = c2[2:]
            gid = s + r
            valid = (gid >= r_lo) & (gid >= base) & (gid < r_hi)
            b_r = _scalar_at(bb, r)
            m_r = _scalar_at(mb, r)
            xs = [xb[r, pl.ds(cc * L, L)] for cc in range(NCOL)]
            dv = zero16
            for cc in range(NCOL):
                dv = dv + xs[cc] * wregs[cc]
            z = jnp.sum(dv)
            zv = jnp.full((L,), z, jnp.float32) + bvec
            gate = 1.0 / (1.0 + jnp.exp(-zv))
            ev = jnp.where(valid & (m_r == 1), jnp.exp(gate), zero16)

            switch = valid & (b_r != cur_g)

            @pl.when(switch & (cur_g >= 0))
            def _():
                _flush(cur_g, denv, accs)

            keep = jnp.where(switch, 0.0, 1.0)
            new_denv = denv * keep + ev
            new_accs = tuple(accs[cc] * keep + xs[cc] * ev
                             for cc in range(NCOL))
            new_g = jnp.where(switch, b_r, cur_g)
            return (new_g, new_denv) + new_accs

        return lax.fori_loop(0, CH, row_body, carry)

    init = (jnp.int32(-1), zero16) + tuple(zero16 for _ in range(NCOL))
    bufs0 = (xbuf0, bidx0, midx0)
    bufs1 = (xbuf1, bidx1, midx1)

    @pl.when(nchp > 0)
    def _():
        _start(0, bufs0, sem0)

    def pair_body(p, carry):
        k0 = 2 * p
        k1 = k0 + 1
        _start(k1, bufs1, sem1)
        _wait(k0, bufs0, sem0)
        carry = _process(k0, bufs0, carry)

        @pl.when(k0 + 2 < nchp)
        def _():
            _start(k0 + 2, bufs0, sem0)

        _wait(k1, bufs1, sem1)
        return _process(k1, bufs1, carry)

    carry = lax.fori_loop(0, nchp // 2, pair_body, init)

    cur_g = carry[0]

    @pl.when(cur_g >= 0)
    def _():
        _flush(cur_g, carry[1], carry[2:])

    pltpu.sync_copy(accbuf, out_hbm.at[pl.ds(g_base, GPW)])


_sc_pool = functools.partial(
    pl.kernel,
    out_type=jax.ShapeDtypeStruct((G, DIM), jnp.float32),
    mesh=plsc.VectorSubcoreMesh(core_axis_name="c", subcore_axis_name="s"),
    scratch_types=[
        pltpu.VMEM((CH, DIM), jnp.float32),
        pltpu.VMEM((CH,), jnp.int32),
        pltpu.VMEM((CH,), jnp.int32),
        pltpu.VMEM((CH, DIM), jnp.float32),
        pltpu.VMEM((CH,), jnp.int32),
        pltpu.VMEM((CH,), jnp.int32),
        pltpu.VMEM((DIM,), jnp.float32),
        pltpu.VMEM((L,), jnp.float32),
        pltpu.VMEM((BPAD,), jnp.int32),
        pltpu.VMEM((GPW, DIM), jnp.float32),
        pltpu.SemaphoreType.DMA,
        pltpu.SemaphoreType.DMA,
    ],
    compiler_params=pltpu.CompilerParams(needs_layout_passes=False),
)(_sc_body)


@jax.jit
def kernel(x, batch, repeat_unit_mask, gate_w, gate_b):
    batch = batch.astype(jnp.int32)
    w = gate_w.reshape(DIM).astype(jnp.float32)
    b = jnp.broadcast_to(gate_b.astype(jnp.float32), (L,))
    edges = jnp.arange(0, G + GPW, GPW, dtype=jnp.int32)
    bounds = jnp.searchsorted(batch, edges).astype(jnp.int32)
    bounds = jnp.pad(bounds, (0, BPAD - bounds.shape[0]))
    return _sc_pool(x, batch, repeat_unit_mask.astype(jnp.int32), w, b,
                    bounds)
